# baseline (speedup 1.0000x reference)
import jax
import jax.numpy as jnp
from jax.experimental import pallas as pl

N = 128
B = 256
NE = N - 1
BI = 16


def _body(ib_x_ref, xt_ref, rel_ref, rinv_ref, ea_ref, ep_ref):
    ib = pl.program_id(1)

    xt = xt_ref[0]
    px_r = xt[0:1, :]
    py_r = xt[1:2, :]
    vx_r = xt[2:3, :]
    vy_r = xt[3:4, :]
    inv_r = jax.lax.rsqrt(jnp.maximum(vx_r * vx_r + vy_r * vy_r, 1e-30))
    c_r = vx_r * inv_r
    s_r = vy_r * inv_r
    cx_r = c_r * vx_r + s_r * vy_r
    cy_r = c_r * vy_r - s_r * vx_r

    z = jnp.zeros_like(c_r)
    rel_ref[0] = jnp.concatenate([z, z, cx_r, cy_r], axis=0).T
    rinv_ref[0] = jnp.stack(
        [jnp.concatenate([c_r, s_r], axis=0).T,
         jnp.concatenate([-s_r, c_r], axis=0).T], axis=-2)

    xs = ib_x_ref[0]
    px_s = xs[:, 0:1]
    py_s = xs[:, 1:2]
    vx_s = xs[:, 2:3]
    vy_s = xs[:, 3:4]

    dx = px_s - px_r
    dy = py_s - py_r
    rot_x = c_r * dx + s_r * dy
    rot_y = c_r * dy - s_r * dx
    r2 = dx * dx + dy * dy
    inv = jax.lax.rsqrt(jnp.maximum(r2, 1e-30))
    r = r2 * inv
    cphi = rot_x * inv
    sphi = rot_y * inv
    rvx = c_r * vx_s + s_r * vy_s
    rvy = c_r * vy_s - s_r * vx_s
    zz = jnp.zeros_like(dx)
    cxb = jnp.broadcast_to(cx_r, dx.shape)
    cyb = jnp.broadcast_to(cy_r, dx.shape)

    row = jax.lax.broadcasted_iota(jnp.int32, (BI, NE), 0) + ib * BI
    col = jax.lax.broadcasted_iota(jnp.int32, (BI, NE), 1)
    keep_lo = col < row

    def sel(g):
        return jnp.where(keep_lo, g[:, :NE], g[:, 1:])

    f = [sel(g) for g in
         (rot_x, rot_y, r, cphi, sphi, rvx, rvy, zz, zz, cxb, cyb)]
    ea_ref[0] = jnp.stack(f, axis=-1)
    ep_ref[0] = jnp.stack(f[:3], axis=-1)


def kernel(x):
    xt = jnp.transpose(x, (0, 2, 1))
    rel, rinv, ea4, ep4 = pl.pallas_call(
        _body,
        grid=(B, N // BI),
        in_specs=[
            pl.BlockSpec((1, BI, 4), lambda b, ib: (b, ib, 0)),
            pl.BlockSpec((1, 4, N), lambda b, ib: (b, 0, 0)),
        ],
        out_specs=[
            pl.BlockSpec((1, N, 4), lambda b, ib: (b, 0, 0)),
            pl.BlockSpec((1, N, 2, 2), lambda b, ib: (b, 0, 0, 0)),
            pl.BlockSpec((1, BI, NE, 11), lambda b, ib: (b, ib, 0, 0)),
            pl.BlockSpec((1, BI, NE, 3), lambda b, ib: (b, ib, 0, 0)),
        ],
        out_shape=[
            jax.ShapeDtypeStruct((B, N, 4), jnp.float32),
            jax.ShapeDtypeStruct((B, N, 2, 2), jnp.float32),
            jax.ShapeDtypeStruct((B, N, NE, 11), jnp.float32),
            jax.ShapeDtypeStruct((B, N, NE, 3), jnp.float32),
        ],
    )(x, xt)
    return (rel, rinv,
            ea4.reshape(B, N * NE, 11),
            ep4.reshape(B, N * NE, 3))
